# baseline (speedup 1.0000x reference)
import functools

import jax
import jax.numpy as jnp
from jax import lax
from jax.experimental import pallas as pl
from jax.experimental.pallas import tpu as pltpu
from jax.experimental.pallas import tpu_sc as plsc

N = 10000
E = 160000
DIM = 32
EDGE_DIM = 16
STEPS = 6

NP = 10240
EP = 163840
W_EDGES = 5120
CHUNK = 128
N_CHUNKS = W_EDGES // CHUNK
MACRO = 4
CPM = N_CHUNKS // MACRO
ROWS_MACRO = CPM * CHUNK
ROWS_TILE = NP // 16

BE = 2048
BN = 2000

_MESH = plsc.VectorSubcoreMesh(core_axis_name="c", subcore_axis_name="s")
_SC_PARAMS = pltpu.CompilerParams(use_tc_tiling_on_sc=False)


@functools.partial(
    pl.kernel,
    mesh=_MESH,
    out_type=jax.ShapeDtypeStruct((EP, DIM), jnp.float32),
    scratch_types=[
        pltpu.VMEM((N_CHUNKS, CHUNK), jnp.int32),
        pltpu.VMEM((ROWS_MACRO, DIM), jnp.float32),
        pltpu.SemaphoreType.DMA,
    ],
    compiler_params=_SC_PARAMS,
)
def _sc_gather(table_hbm, idx2_hbm, out_hbm, idx_v, rows_v, sem):
    cid = lax.axis_index("c")
    sid = lax.axis_index("s")
    wid = sid * 2 + cid
    pltpu.sync_copy(idx2_hbm.at[pl.ds(wid * N_CHUNKS, N_CHUNKS)], idx_v)
    for m in range(MACRO):
        copies = [
            pltpu.async_copy(
                table_hbm.at[idx_v.at[m * CPM + j]],
                rows_v.at[pl.ds(j * CHUNK, CHUNK)],
                sem,
            )
            for j in range(CPM)
        ]
        for cp in copies:
            cp.wait()
        pltpu.sync_copy(
            rows_v,
            out_hbm.at[pl.ds(wid * W_EDGES + m * ROWS_MACRO, ROWS_MACRO)],
        )


@functools.partial(
    pl.kernel,
    mesh=_MESH,
    out_type=jax.ShapeDtypeStruct((2, NP, DIM), jnp.float32),
    scratch_types=[
        pltpu.VMEM((N_CHUNKS, CHUNK), jnp.int32),
        pltpu.VMEM((ROWS_MACRO, DIM), jnp.float32),
        pltpu.VMEM_SHARED((NP, DIM), jnp.float32),
    ],
    compiler_params=_SC_PARAMS,
)
def _sc_scatter(vals_hbm, idx2_hbm, zeros_hbm, out_hbm, idx_v, vals_v, shared):
    cid = lax.axis_index("c")
    sid = lax.axis_index("s")
    wid = sid * 2 + cid
    pltpu.sync_copy(
        zeros_hbm.at[pl.ds(sid * ROWS_TILE, ROWS_TILE)],
        shared.at[pl.ds(sid * ROWS_TILE, ROWS_TILE)],
    )
    plsc.subcore_barrier()
    pltpu.sync_copy(idx2_hbm.at[pl.ds(wid * N_CHUNKS, N_CHUNKS)], idx_v)
    for m in range(MACRO):
        pltpu.sync_copy(
            vals_hbm.at[pl.ds(wid * W_EDGES + m * ROWS_MACRO, ROWS_MACRO)],
            vals_v,
        )
        for j in range(CPM):
            pltpu.sync_copy(
                vals_v.at[pl.ds(j * CHUNK, CHUNK)],
                shared.at[idx_v.at[m * CPM + j]],
                add=True,
            )
    plsc.subcore_barrier()
    pltpu.sync_copy(
        shared.at[pl.ds(sid * ROWS_TILE, ROWS_TILE)],
        out_hbm.at[cid, pl.ds(sid * ROWS_TILE, ROWS_TILE)],
    )


def _embed_body(x_ref, fct_ref, fcb_ref, out_ref):
    out_ref[...] = jax.nn.relu(
        jnp.dot(x_ref[...], fct_ref[...], preferred_element_type=jnp.float32)
        + fcb_ref[...]
    )


def _msg_body(xj_ref, ea_ref, w1t_ref, b1_ref, w2b_ref, b2m_ref, out_ref):
    ea = ea_ref[...]
    xj = xj_ref[...]
    hidden = jax.nn.relu(
        jnp.dot(ea, w1t_ref[...], preferred_element_type=jnp.float32) + b1_ref[...]
    )
    p2 = jnp.dot(xj, w2b_ref[...], preferred_element_type=jnp.float32)
    acc = jnp.dot(xj, b2m_ref[...], preferred_element_type=jnp.float32)
    for k in range(DIM):
        acc = acc + hidden[:, k:k + 1] * p2[:, DIM * k:DIM * k + DIM]
    out_ref[...] = acc


def _update_body(aggp_ref, degp_ref, h_ref, root_ref, cb_ref, wih_ref, bih_ref,
                 whh_ref, bhh_ref, out_ref):
    a = aggp_ref[0] + aggp_ref[1]
    d = jnp.maximum(degp_ref[0] + degp_ref[1], 1.0)
    h = h_ref[...]
    conv = a / d + jnp.dot(h, root_ref[...], preferred_element_type=jnp.float32) + cb_ref[...]
    m = jax.nn.relu(conv)
    gi = jnp.dot(m, wih_ref[...], preferred_element_type=jnp.float32) + bih_ref[...]
    gh = jnp.dot(h, whh_ref[...], preferred_element_type=jnp.float32) + bhh_ref[...]
    r = jax.nn.sigmoid(gi[:, 0:DIM] + gh[:, 0:DIM])
    z = jax.nn.sigmoid(gi[:, DIM:2 * DIM] + gh[:, DIM:2 * DIM])
    n = jnp.tanh(gi[:, 2 * DIM:3 * DIM] + r * gh[:, 2 * DIM:3 * DIM])
    out_ref[...] = (1.0 - z) * n + z * h


_embed_call = pl.pallas_call(
    _embed_body,
    out_shape=jax.ShapeDtypeStruct((N, DIM), jnp.float32),
)

_msg_call = pl.pallas_call(
    _msg_body,
    grid=(EP // BE,),
    in_specs=[
        pl.BlockSpec((BE, DIM), lambda i: (i, 0)),
        pl.BlockSpec((BE, EDGE_DIM), lambda i: (i, 0)),
        pl.BlockSpec((EDGE_DIM, DIM), lambda i: (0, 0)),
        pl.BlockSpec((1, DIM), lambda i: (0, 0)),
        pl.BlockSpec((DIM, DIM * DIM), lambda i: (0, 0)),
        pl.BlockSpec((DIM, DIM), lambda i: (0, 0)),
    ],
    out_specs=pl.BlockSpec((BE, DIM), lambda i: (i, 0)),
    out_shape=jax.ShapeDtypeStruct((EP, DIM), jnp.float32),
)

_upd_call = pl.pallas_call(
    _update_body,
    grid=(N // BN,),
    in_specs=[
        pl.BlockSpec((2, BN, DIM), lambda i: (0, i, 0)),
        pl.BlockSpec((2, BN, DIM), lambda i: (0, i, 0)),
        pl.BlockSpec((BN, DIM), lambda i: (i, 0)),
        pl.BlockSpec((DIM, DIM), lambda i: (0, 0)),
        pl.BlockSpec((1, DIM), lambda i: (0, 0)),
        pl.BlockSpec((DIM, 3 * DIM), lambda i: (0, 0)),
        pl.BlockSpec((1, 3 * DIM), lambda i: (0, 0)),
        pl.BlockSpec((DIM, 3 * DIM), lambda i: (0, 0)),
        pl.BlockSpec((1, 3 * DIM), lambda i: (0, 0)),
    ],
    out_specs=pl.BlockSpec((BN, DIM), lambda i: (i, 0)),
    out_shape=jax.ShapeDtypeStruct((N, DIM), jnp.float32),
)


def kernel(x, edge_index, edge_attr, fc_W, fc_b, root, conv_bias,
           en_W1, en_b1, en_W2, en_b2, gru_Wih, gru_Whh, gru_bih, gru_bhh):
    src = edge_index[0].astype(jnp.int32)
    dst = edge_index[1].astype(jnp.int32)
    pad_e = EP - E
    src2 = jnp.concatenate([src, jnp.zeros((pad_e,), jnp.int32)]).reshape(EP // CHUNK, CHUNK)
    dst2 = jnp.concatenate([dst, jnp.full((pad_e,), N, jnp.int32)]).reshape(EP // CHUNK, CHUNK)
    ea_p = jnp.concatenate([edge_attr, jnp.zeros((pad_e, EDGE_DIM), jnp.float32)])

    w1t = en_W1.T
    b1 = en_b1.reshape(1, DIM)
    w2b = en_W2.reshape(DIM, DIM, DIM).transpose(0, 2, 1).reshape(DIM, DIM * DIM)
    b2m = en_b2.reshape(DIM, DIM)
    wih = gru_Wih.T
    whh = gru_Whh.T
    bih = gru_bih.reshape(1, 3 * DIM)
    bhh = gru_bhh.reshape(1, 3 * DIM)
    cb = conv_bias.reshape(1, DIM)

    zeros_np = jnp.zeros((NP, DIM), jnp.float32)
    ones_ep = jnp.ones((EP, DIM), jnp.float32)

    h = _embed_call(x, fc_W.T, fc_b.reshape(1, DIM))
    degp = _sc_scatter(ones_ep, dst2, zeros_np)
    for _ in range(STEPS):
        xj = _sc_gather(h, src2)
        msg = _msg_call(xj, ea_p, w1t, b1, w2b, b2m)
        aggp = _sc_scatter(msg, dst2, zeros_np)
        h = _upd_call(aggp, degp, h, root, cb, wih, bih, whh, bhh)
    return h


# speedup vs baseline: 4.0041x; 4.0041x over previous
import functools

import jax
import jax.numpy as jnp
from jax import lax
from jax.experimental import pallas as pl
from jax.experimental.pallas import tpu as pltpu
from jax.experimental.pallas import tpu_sc as plsc

N = 10000
E = 160000
DIM = 32
EDGE_DIM = 16
STEPS = 6

NP = 10240
EP = 163840
W_EDGES = 5120
CHUNK = 128
N_CHUNKS = W_EDGES // CHUNK
MACRO = 4
CPM = N_CHUNKS // MACRO
ROWS_MACRO = CPM * CHUNK
ROWS_TILE = NP // 16

BE = 2048
BN = 2000

_MESH = plsc.VectorSubcoreMesh(core_axis_name="c", subcore_axis_name="s")
_SC_PARAMS = pltpu.CompilerParams(use_tc_tiling_on_sc=False)


@functools.partial(
    pl.kernel,
    mesh=_MESH,
    out_type=jax.ShapeDtypeStruct((EP, DIM), jnp.float32),
    scratch_types=[
        pltpu.VMEM((N_CHUNKS, CHUNK), jnp.int32),
        pltpu.VMEM((ROWS_MACRO, DIM), jnp.float32),
        pltpu.SemaphoreType.DMA,
    ],
    compiler_params=_SC_PARAMS,
)
def _sc_gather(table_hbm, idx2_hbm, out_hbm, idx_v, rows_v, sem):
    cid = lax.axis_index("c")
    sid = lax.axis_index("s")
    wid = sid * 2 + cid
    pltpu.sync_copy(idx2_hbm.at[pl.ds(wid * N_CHUNKS, N_CHUNKS)], idx_v)
    for m in range(MACRO):
        copies = [
            pltpu.async_copy(
                table_hbm.at[idx_v.at[m * CPM + j]],
                rows_v.at[pl.ds(j * CHUNK, CHUNK)],
                sem,
            )
            for j in range(CPM)
        ]
        for cp in copies:
            cp.wait()
        pltpu.sync_copy(
            rows_v,
            out_hbm.at[pl.ds(wid * W_EDGES + m * ROWS_MACRO, ROWS_MACRO)],
        )


@functools.partial(
    pl.kernel,
    mesh=_MESH,
    out_type=jax.ShapeDtypeStruct((2, NP, DIM), jnp.float32),
    scratch_types=[
        pltpu.VMEM((N_CHUNKS, CHUNK), jnp.int32),
        pltpu.VMEM((ROWS_MACRO, DIM), jnp.float32),
        pltpu.VMEM_SHARED((NP, DIM), jnp.float32),
    ],
    compiler_params=_SC_PARAMS,
)
def _sc_scatter(vals_hbm, idx2_hbm, zeros_hbm, out_hbm, idx_v, vals_v, shared):
    cid = lax.axis_index("c")
    sid = lax.axis_index("s")
    wid = sid * 2 + cid
    pltpu.sync_copy(
        zeros_hbm.at[pl.ds(sid * ROWS_TILE, ROWS_TILE)],
        shared.at[pl.ds(sid * ROWS_TILE, ROWS_TILE)],
    )
    plsc.subcore_barrier()
    pltpu.sync_copy(idx2_hbm.at[pl.ds(wid * N_CHUNKS, N_CHUNKS)], idx_v)
    for m in range(MACRO):
        pltpu.sync_copy(
            vals_hbm.at[pl.ds(wid * W_EDGES + m * ROWS_MACRO, ROWS_MACRO)],
            vals_v,
        )
        for j in range(CPM):
            pltpu.sync_copy(
                vals_v.at[pl.ds(j * CHUNK, CHUNK)],
                shared.at[idx_v.at[m * CPM + j]],
                add=True,
            )
    plsc.subcore_barrier()
    pltpu.sync_copy(
        shared.at[pl.ds(sid * ROWS_TILE, ROWS_TILE)],
        out_hbm.at[cid, pl.ds(sid * ROWS_TILE, ROWS_TILE)],
    )


def _embed_body(x_ref, fct_ref, fcb_ref, out_ref):
    out_ref[...] = jax.nn.relu(
        jnp.dot(x_ref[...], fct_ref[...], preferred_element_type=jnp.float32)
        + fcb_ref[...]
    )


def _msg_body(xj_ref, eat_ref, w1_ref, b1c_ref, m2_ref, b2t_ref, out_ref):
    xjt = xj_ref[...].T
    ht = jax.nn.relu(
        jnp.dot(w1_ref[...], eat_ref[...], preferred_element_type=jnp.float32)
        + b1c_ref[...]
    )
    at = jnp.concatenate(
        [ht[k:k + 1, :] * xjt for k in range(DIM)], axis=0
    )
    msgt = (
        jnp.dot(m2_ref[...], at, preferred_element_type=jnp.float32)
        + jnp.dot(b2t_ref[...], xjt, preferred_element_type=jnp.float32)
    )
    out_ref[...] = msgt.T


def _update_body(aggp_ref, degp_ref, h_ref, root_ref, cb_ref, wih_ref, bih_ref,
                 whh_ref, bhh_ref, out_ref):
    a = aggp_ref[0] + aggp_ref[1]
    d = jnp.maximum(degp_ref[0] + degp_ref[1], 1.0)
    h = h_ref[...]
    conv = a / d + jnp.dot(h, root_ref[...], preferred_element_type=jnp.float32) + cb_ref[...]
    m = jax.nn.relu(conv)
    gi = jnp.dot(m, wih_ref[...], preferred_element_type=jnp.float32) + bih_ref[...]
    gh = jnp.dot(h, whh_ref[...], preferred_element_type=jnp.float32) + bhh_ref[...]
    r = jax.nn.sigmoid(gi[:, 0:DIM] + gh[:, 0:DIM])
    z = jax.nn.sigmoid(gi[:, DIM:2 * DIM] + gh[:, DIM:2 * DIM])
    n = jnp.tanh(gi[:, 2 * DIM:3 * DIM] + r * gh[:, 2 * DIM:3 * DIM])
    out_ref[...] = (1.0 - z) * n + z * h


_embed_call = pl.pallas_call(
    _embed_body,
    out_shape=jax.ShapeDtypeStruct((N, DIM), jnp.float32),
)

_msg_call = pl.pallas_call(
    _msg_body,
    grid=(EP // BE,),
    in_specs=[
        pl.BlockSpec((BE, DIM), lambda i: (i, 0)),
        pl.BlockSpec((EDGE_DIM, BE), lambda i: (0, i)),
        pl.BlockSpec((DIM, EDGE_DIM), lambda i: (0, 0)),
        pl.BlockSpec((DIM, 1), lambda i: (0, 0)),
        pl.BlockSpec((DIM, DIM * DIM), lambda i: (0, 0)),
        pl.BlockSpec((DIM, DIM), lambda i: (0, 0)),
    ],
    out_specs=pl.BlockSpec((BE, DIM), lambda i: (i, 0)),
    out_shape=jax.ShapeDtypeStruct((EP, DIM), jnp.float32),
)

_upd_call = pl.pallas_call(
    _update_body,
    grid=(N // BN,),
    in_specs=[
        pl.BlockSpec((2, BN, DIM), lambda i: (0, i, 0)),
        pl.BlockSpec((2, BN, DIM), lambda i: (0, i, 0)),
        pl.BlockSpec((BN, DIM), lambda i: (i, 0)),
        pl.BlockSpec((DIM, DIM), lambda i: (0, 0)),
        pl.BlockSpec((1, DIM), lambda i: (0, 0)),
        pl.BlockSpec((DIM, 3 * DIM), lambda i: (0, 0)),
        pl.BlockSpec((1, 3 * DIM), lambda i: (0, 0)),
        pl.BlockSpec((DIM, 3 * DIM), lambda i: (0, 0)),
        pl.BlockSpec((1, 3 * DIM), lambda i: (0, 0)),
    ],
    out_specs=pl.BlockSpec((BN, DIM), lambda i: (i, 0)),
    out_shape=jax.ShapeDtypeStruct((N, DIM), jnp.float32),
)


def kernel(x, edge_index, edge_attr, fc_W, fc_b, root, conv_bias,
           en_W1, en_b1, en_W2, en_b2, gru_Wih, gru_Whh, gru_bih, gru_bhh):
    src = edge_index[0].astype(jnp.int32)
    dst = edge_index[1].astype(jnp.int32)
    pad_e = EP - E
    src2 = jnp.concatenate([src, jnp.zeros((pad_e,), jnp.int32)]).reshape(EP // CHUNK, CHUNK)
    dst2 = jnp.concatenate([dst, jnp.full((pad_e,), N, jnp.int32)]).reshape(EP // CHUNK, CHUNK)
    eat = jnp.concatenate(
        [edge_attr, jnp.zeros((pad_e, EDGE_DIM), jnp.float32)]).T

    b1c = en_b1.reshape(DIM, 1)
    m2 = en_W2.reshape(DIM, DIM, DIM).transpose(1, 2, 0).reshape(DIM, DIM * DIM)
    b2t = en_b2.reshape(DIM, DIM).T
    wih = gru_Wih.T
    whh = gru_Whh.T
    bih = gru_bih.reshape(1, 3 * DIM)
    bhh = gru_bhh.reshape(1, 3 * DIM)
    cb = conv_bias.reshape(1, DIM)

    zeros_np = jnp.zeros((NP, DIM), jnp.float32)
    ones_ep = jnp.ones((EP, DIM), jnp.float32)

    h = _embed_call(x, fc_W.T, fc_b.reshape(1, DIM))
    degp = _sc_scatter(ones_ep, dst2, zeros_np)
    for _ in range(STEPS):
        xj = _sc_gather(h, src2)
        msg = _msg_call(xj, eat, en_W1, b1c, m2, b2t)
        aggp = _sc_scatter(msg, dst2, zeros_np)
        h = _upd_call(aggp, degp, h, root, cb, wih, bih, whh, bhh)
    return h


# speedup vs baseline: 6.2039x; 1.5494x over previous
import functools

import jax
import jax.numpy as jnp
from jax import lax
from jax.experimental import pallas as pl
from jax.experimental.pallas import tpu as pltpu
from jax.experimental.pallas import tpu_sc as plsc

N = 10000
E = 160000
DIM = 32
EDGE_DIM = 16
STEPS = 6

NP = 10240
EP = 163840
W_EDGES = 5120
CHUNK = 128
N_CHUNKS = W_EDGES // CHUNK
MACRO = 4
CPM = N_CHUNKS // MACRO
ROWS_MACRO = CPM * CHUNK
ROWS_TILE = NP // 16

BE = 2048
BN = 2000

_MESH = plsc.VectorSubcoreMesh(core_axis_name="c", subcore_axis_name="s")
_SC_PARAMS = pltpu.CompilerParams(use_tc_tiling_on_sc=False)


@functools.partial(
    pl.kernel,
    mesh=_MESH,
    out_type=jax.ShapeDtypeStruct((EP, DIM), jnp.float32),
    scratch_types=[
        pltpu.VMEM((N_CHUNKS, CHUNK), jnp.int32),
        pltpu.VMEM((ROWS_MACRO, DIM), jnp.float32),
        pltpu.SemaphoreType.DMA,
    ],
    compiler_params=_SC_PARAMS,
)
def _sc_gather(table_hbm, idx2_hbm, out_hbm, idx_v, rows_v, sem):
    cid = lax.axis_index("c")
    sid = lax.axis_index("s")
    wid = sid * 2 + cid
    pltpu.sync_copy(idx2_hbm.at[pl.ds(wid * N_CHUNKS, N_CHUNKS)], idx_v)
    for m in range(MACRO):
        copies = [
            pltpu.async_copy(
                table_hbm.at[idx_v.at[m * CPM + j]],
                rows_v.at[pl.ds(j * CHUNK, CHUNK)],
                sem,
            )
            for j in range(CPM)
        ]
        for cp in copies:
            cp.wait()
        pltpu.sync_copy(
            rows_v,
            out_hbm.at[pl.ds(wid * W_EDGES + m * ROWS_MACRO, ROWS_MACRO)],
        )


@functools.partial(
    pl.kernel,
    mesh=_MESH,
    out_type=jax.ShapeDtypeStruct((2, NP, DIM), jnp.float32),
    scratch_types=[
        pltpu.VMEM((N_CHUNKS, CHUNK), jnp.int32),
        pltpu.VMEM((ROWS_MACRO, DIM), jnp.float32),
        pltpu.VMEM_SHARED((NP, DIM), jnp.float32),
    ],
    compiler_params=_SC_PARAMS,
)
def _sc_scatter(vals_hbm, idx2_hbm, zeros_hbm, out_hbm, idx_v, vals_v, shared):
    cid = lax.axis_index("c")
    sid = lax.axis_index("s")
    wid = sid * 2 + cid
    pltpu.sync_copy(
        zeros_hbm.at[pl.ds(sid * ROWS_TILE, ROWS_TILE)],
        shared.at[pl.ds(sid * ROWS_TILE, ROWS_TILE)],
    )
    plsc.subcore_barrier()
    pltpu.sync_copy(idx2_hbm.at[pl.ds(wid * N_CHUNKS, N_CHUNKS)], idx_v)
    for m in range(MACRO):
        pltpu.sync_copy(
            vals_hbm.at[pl.ds(wid * W_EDGES + m * ROWS_MACRO, ROWS_MACRO)],
            vals_v,
        )
        for j in range(CPM):
            pltpu.sync_copy(
                vals_v.at[pl.ds(j * CHUNK, CHUNK)],
                shared.at[idx_v.at[m * CPM + j]],
                add=True,
            )
    plsc.subcore_barrier()
    pltpu.sync_copy(
        shared.at[pl.ds(sid * ROWS_TILE, ROWS_TILE)],
        out_hbm.at[cid, pl.ds(sid * ROWS_TILE, ROWS_TILE)],
    )


def _embed_body(x_ref, fct_ref, fcb_ref, out_ref):
    out_ref[...] = jax.nn.relu(
        jnp.dot(x_ref[...], fct_ref[...], preferred_element_type=jnp.float32)
        + fcb_ref[...]
    )


PACK = 4
PACK_G = BE // PACK


def _msg_body(xjp_ref, eat_ref, w1_ref, b1c_ref, m2_ref, b2t_ref, out_ref):
    pt = xjp_ref[...].T
    xjt = jnp.concatenate(
        [pt[DIM * c:DIM * (c + 1), :] for c in range(PACK)], axis=1
    )
    ht = jax.nn.relu(
        jnp.dot(w1_ref[...], eat_ref[...], preferred_element_type=jnp.float32)
        + b1c_ref[...]
    )
    at = jnp.concatenate(
        [ht[k:k + 1, :] * xjt for k in range(DIM)], axis=0
    )
    msgt = (
        jnp.dot(m2_ref[...], at, preferred_element_type=jnp.float32)
        + jnp.dot(b2t_ref[...], xjt, preferred_element_type=jnp.float32)
    )
    mp = jnp.concatenate(
        [msgt[:, PACK_G * c:PACK_G * (c + 1)] for c in range(PACK)], axis=0
    )
    out_ref[...] = mp.T


def _update_body(aggp_ref, degp_ref, h_ref, root_ref, cb_ref, wih_ref, bih_ref,
                 whh_ref, bhh_ref, out_ref):
    a = aggp_ref[0] + aggp_ref[1]
    d = jnp.maximum(degp_ref[0] + degp_ref[1], 1.0)
    h = h_ref[...]
    conv = a / d + jnp.dot(h, root_ref[...], preferred_element_type=jnp.float32) + cb_ref[...]
    m = jax.nn.relu(conv)
    gi = jnp.dot(m, wih_ref[...], preferred_element_type=jnp.float32) + bih_ref[...]
    gh = jnp.dot(h, whh_ref[...], preferred_element_type=jnp.float32) + bhh_ref[...]
    r = jax.nn.sigmoid(gi[:, 0:DIM] + gh[:, 0:DIM])
    z = jax.nn.sigmoid(gi[:, DIM:2 * DIM] + gh[:, DIM:2 * DIM])
    n = jnp.tanh(gi[:, 2 * DIM:3 * DIM] + r * gh[:, 2 * DIM:3 * DIM])
    out_ref[...] = (1.0 - z) * n + z * h


_embed_call = pl.pallas_call(
    _embed_body,
    out_shape=jax.ShapeDtypeStruct((N, DIM), jnp.float32),
)

_msg_call = pl.pallas_call(
    _msg_body,
    grid=(EP // BE,),
    in_specs=[
        pl.BlockSpec((PACK_G, 128), lambda i: (i, 0)),
        pl.BlockSpec((EDGE_DIM, BE), lambda i: (0, i)),
        pl.BlockSpec((DIM, EDGE_DIM), lambda i: (0, 0)),
        pl.BlockSpec((DIM, 1), lambda i: (0, 0)),
        pl.BlockSpec((DIM, DIM * DIM), lambda i: (0, 0)),
        pl.BlockSpec((DIM, DIM), lambda i: (0, 0)),
    ],
    out_specs=pl.BlockSpec((PACK_G, 128), lambda i: (i, 0)),
    out_shape=jax.ShapeDtypeStruct((EP // PACK, 128), jnp.float32),
)

_upd_call = pl.pallas_call(
    _update_body,
    grid=(N // BN,),
    in_specs=[
        pl.BlockSpec((2, BN, DIM), lambda i: (0, i, 0)),
        pl.BlockSpec((2, BN, DIM), lambda i: (0, i, 0)),
        pl.BlockSpec((BN, DIM), lambda i: (i, 0)),
        pl.BlockSpec((DIM, DIM), lambda i: (0, 0)),
        pl.BlockSpec((1, DIM), lambda i: (0, 0)),
        pl.BlockSpec((DIM, 3 * DIM), lambda i: (0, 0)),
        pl.BlockSpec((1, 3 * DIM), lambda i: (0, 0)),
        pl.BlockSpec((DIM, 3 * DIM), lambda i: (0, 0)),
        pl.BlockSpec((1, 3 * DIM), lambda i: (0, 0)),
    ],
    out_specs=pl.BlockSpec((BN, DIM), lambda i: (i, 0)),
    out_shape=jax.ShapeDtypeStruct((N, DIM), jnp.float32),
)


def kernel(x, edge_index, edge_attr, fc_W, fc_b, root, conv_bias,
           en_W1, en_b1, en_W2, en_b2, gru_Wih, gru_Whh, gru_bih, gru_bhh):
    src = edge_index[0].astype(jnp.int32)
    dst = edge_index[1].astype(jnp.int32)
    pad_e = EP - E
    src2 = jnp.concatenate([src, jnp.zeros((pad_e,), jnp.int32)]).reshape(EP // CHUNK, CHUNK)
    dst2 = jnp.concatenate([dst, jnp.full((pad_e,), N, jnp.int32)]).reshape(EP // CHUNK, CHUNK)
    ea_p = jnp.concatenate(
        [edge_attr, jnp.zeros((pad_e, EDGE_DIM), jnp.float32)])
    eat = (ea_p.reshape(EP // BE, PACK_G, PACK, EDGE_DIM).transpose(0, 2, 1, 3)
           .reshape(EP, EDGE_DIM).T)

    b1c = en_b1.reshape(DIM, 1)
    m2 = en_W2.reshape(DIM, DIM, DIM).transpose(1, 2, 0).reshape(DIM, DIM * DIM)
    b2t = en_b2.reshape(DIM, DIM).T
    wih = gru_Wih.T
    whh = gru_Whh.T
    bih = gru_bih.reshape(1, 3 * DIM)
    bhh = gru_bhh.reshape(1, 3 * DIM)
    cb = conv_bias.reshape(1, DIM)

    zeros_np = jnp.zeros((NP, DIM), jnp.float32)
    ones_ep = jnp.ones((EP, DIM), jnp.float32)

    h = _embed_call(x, fc_W.T, fc_b.reshape(1, DIM))
    degp = _sc_scatter(ones_ep, dst2, zeros_np)
    for _ in range(STEPS):
        xj = _sc_gather(h, src2)
        msg_p = _msg_call(xj.reshape(EP // PACK, 128), eat, en_W1, b1c, m2, b2t)
        aggp = _sc_scatter(msg_p.reshape(EP, DIM), dst2, zeros_np)
        h = _upd_call(aggp, degp, h, root, cb, wih, bih, whh, bhh)
    return h
